# baseline (speedup 1.0000x reference)
import functools

import jax
import jax.numpy as jnp
from jax import lax
from jax.experimental import pallas as pl
from jax.experimental.pallas import tpu as pltpu, tpu_sc as plsc

N = 10000
E = 320000
D = 128
H1 = 16

NC = 2
NS = 16
NW = NC * NS
EPW = E // NW
CH = 256
EPWP = 10240
NCHUNK = EPWP // CH
NBUF = 4
NSTEP = NCHUNK // NBUF
NPAD = 10240
RPT = NPAD // NS
PAD_DST = NPAD - 8


def _agg_body(p_hbm, src_hbm, dst_hbm, w_hbm, out_hbm,
              src_v, dst_v, w_v, gbuf, sbuf, stage_v, acc_sh, gsem, ssem):
    c = lax.axis_index("c")
    s = lax.axis_index("s")
    wid = c * NS + s

    pltpu.sync_copy(src_hbm.at[wid], src_v)
    pltpu.sync_copy(dst_hbm.at[wid], dst_v)
    pltpu.sync_copy(w_hbm.at[wid], w_v)

    def _zero(i, carry):
        stage_v[i] = jnp.zeros((16,), jnp.float32)
        return carry
    lax.fori_loop(0, RPT, _zero, 0, unroll=8)
    pltpu.sync_copy(stage_v, acc_sh.at[pl.ds(s * RPT, RPT)])
    plsc.subcore_barrier()

    def _gwait(b):
        pltpu.make_async_copy(p_hbm.at[pl.ds(0, CH)], gbuf.at[b],
                              gsem.at[b]).wait()

    def _swait(b):
        pltpu.make_async_copy(p_hbm.at[pl.ds(0, CH)], sbuf.at[b],
                              ssem.at[b]).wait()

    for b in range(NBUF):
        pltpu.async_copy(p_hbm.at[src_v.at[b]], gbuf.at[b], gsem.at[b])

    def _step(i, carry):
        for b in range(NBUF):
            j = i * NBUF + b
            _gwait(b)

            @pl.when(i > 0)
            def _():
                _swait(b)

            def _grp(g, carry2):
                wv16 = w_v[pl.ds(pl.multiple_of(j * CH + g * 16, 16), 16)] + 1.0
                base = g * 16
                for r in range(16):
                    sbuf[b, base + r] = gbuf[b, base + r] * wv16[r]
                return carry2
            lax.fori_loop(0, CH // 16, _grp, 0, unroll=2)

            pltpu.async_copy(sbuf.at[b], acc_sh.at[dst_v.at[j]],
                             ssem.at[b], add=True)

            @pl.when(i < NSTEP - 1)
            def _():
                pltpu.async_copy(p_hbm.at[src_v.at[j + NBUF]], gbuf.at[b],
                                 gsem.at[b])
        return carry
    lax.fori_loop(0, NSTEP, _step, 0)

    for b in range(NBUF):
        _swait(b)
    plsc.subcore_barrier()

    pltpu.sync_copy(acc_sh.at[pl.ds(s * RPT, RPT)],
                    out_hbm.at[c, pl.ds(s * RPT, RPT)])


_agg = functools.partial(
    pl.kernel,
    out_type=jax.ShapeDtypeStruct((NC, NPAD, H1), jnp.float32),
    mesh=plsc.VectorSubcoreMesh(core_axis_name="c", subcore_axis_name="s"),
    compiler_params=pltpu.CompilerParams(use_tc_tiling_on_sc=False),
    scratch_types=[
        pltpu.VMEM((NCHUNK, CH), jnp.int32),
        pltpu.VMEM((NCHUNK, CH), jnp.int32),
        pltpu.VMEM((EPWP,), jnp.float32),
        pltpu.VMEM((NBUF, CH, H1), jnp.float32),
        pltpu.VMEM((NBUF, CH, H1), jnp.float32),
        pltpu.VMEM((RPT, H1), jnp.float32),
        pltpu.VMEM_SHARED((NPAD, H1), jnp.float32),
        pltpu.SemaphoreType.DMA((NBUF,)),
        pltpu.SemaphoreType.DMA((NBUF,)),
    ],
)(_agg_body)


def _proj_body(x_ref, wt_ref, o_ref):
    o_ref[...] = jnp.dot(x_ref[...], wt_ref[...],
                         preferred_element_type=jnp.float32,
                         precision=jax.lax.Precision.HIGHEST)


def _combine_relu_body(p_ref, parts_ref, sw_ref, b_ref, o_ref):
    acc = parts_ref[0] + parts_ref[1]
    o_ref[...] = jnp.maximum(
        p_ref[...] * sw_ref[...] + acc + b_ref[...], 0.0)


def _combine_mm_body(x_ref, parts_ref, sw_ref, wt_ref, b_ref, o_ref):
    h2 = x_ref[...] * sw_ref[...] + parts_ref[0] + parts_ref[1]
    o_ref[...] = jnp.dot(h2, wt_ref[...],
                         preferred_element_type=jnp.float32,
                         precision=jax.lax.Precision.HIGHEST) + b_ref[...]


def _prep_edges(edge_index, edge_weight):
    npad_e = EPWP - EPW
    src = jnp.concatenate(
        [edge_index[0].reshape(NW, EPW),
         jnp.zeros((NW, npad_e), jnp.int32)], axis=1).reshape(NW, NCHUNK, CH)
    dst = jnp.concatenate(
        [edge_index[1].reshape(NW, EPW),
         jnp.full((NW, npad_e), PAD_DST, jnp.int32)],
        axis=1).reshape(NW, NCHUNK, CH)
    w = jnp.concatenate(
        [edge_weight.reshape(NW, EPW),
         jnp.full((NW, npad_e), -1.0, jnp.float32)], axis=1)
    return src, dst, w


def kernel(features, edge_index, edge_weight, self_weight, W1, b1, W2, b2):
    src, dst, w = _prep_edges(edge_index, edge_weight)
    swp1 = self_weight + 1.0

    p1 = pl.pallas_call(
        _proj_body,
        out_shape=jax.ShapeDtypeStruct((N, H1), jnp.float32),
    )(features, W1.T)

    parts1 = _agg(p1, src, dst, w)[:, :N, :]

    x = pl.pallas_call(
        _combine_relu_body,
        out_shape=jax.ShapeDtypeStruct((N, H1), jnp.float32),
    )(p1, parts1, swp1, b1.reshape(1, H1))

    parts2 = _agg(x, src, dst, w)[:, :N, :]

    out = pl.pallas_call(
        _combine_mm_body,
        out_shape=jax.ShapeDtypeStruct((N, W2.shape[0]), jnp.float32),
    )(x, parts2, swp1, W2.T, b2.reshape(1, W2.shape[0]))

    return out


# speedup vs baseline: 1.2708x; 1.2708x over previous
import functools

import jax
import jax.numpy as jnp
from jax import lax
from jax.experimental import pallas as pl
from jax.experimental.pallas import tpu as pltpu, tpu_sc as plsc

N = 10000
E = 320000
D = 128
H1 = 16

NC = 2
NS = 16
NW = NC * NS
EPW = E // NW
CH = 256
EPWP = 10240
NCHUNK = EPWP // CH
NBUF = 4
NSTEP = NCHUNK // NBUF
NPAD = 10240
RPT = NPAD // NS
PAD_DST = NPAD - 8


def _agg_body(p_hbm, src_hbm, dst_hbm, w_hbm, out_hbm,
              src_v, dst_v, w_v, gbuf, sbuf, stage_v, acc_sh, gsem, ssem):
    c = lax.axis_index("c")
    s = lax.axis_index("s")
    wid = c * NS + s

    pltpu.sync_copy(src_hbm.at[wid], src_v)
    pltpu.sync_copy(dst_hbm.at[wid], dst_v)
    pltpu.sync_copy(w_hbm.at[wid], w_v)

    def _zero(i, carry):
        stage_v[i] = jnp.zeros((16,), jnp.float32)
        return carry
    lax.fori_loop(0, RPT, _zero, 0, unroll=8)
    pltpu.sync_copy(stage_v, acc_sh.at[pl.ds(s * RPT, RPT)])
    plsc.subcore_barrier()

    def _gwait(b):
        pltpu.make_async_copy(p_hbm.at[pl.ds(0, CH)], gbuf.at[b],
                              gsem.at[b]).wait()

    def _swait(b):
        pltpu.make_async_copy(p_hbm.at[pl.ds(0, CH)], sbuf.at[b],
                              ssem.at[b]).wait()

    for b in range(NBUF):
        pltpu.async_copy(p_hbm.at[src_v.at[b]], gbuf.at[b], gsem.at[b])

    def _step(i, carry):
        for b in range(NBUF):
            j = i * NBUF + b
            _gwait(b)

            @pl.when(i > 0)
            def _():
                _swait(b)

            def _grp(g, carry2):
                wv16 = w_v[pl.ds(pl.multiple_of(j * CH + g * 16, 16), 16)] + 1.0
                base = g * 16
                for r in range(16):
                    sbuf[b, base + r] = gbuf[b, base + r] * wv16[r]
                return carry2
            lax.fori_loop(0, CH // 16, _grp, 0)

            pltpu.async_copy(sbuf.at[b], acc_sh.at[dst_v.at[j]],
                             ssem.at[b], add=True)

            @pl.when(i < NSTEP - 1)
            def _():
                pltpu.async_copy(p_hbm.at[src_v.at[j + NBUF]], gbuf.at[b],
                                 gsem.at[b])
        return carry
    lax.fori_loop(0, NSTEP, _step, 0)

    for b in range(NBUF):
        _swait(b)
    plsc.subcore_barrier()

    pltpu.sync_copy(acc_sh.at[pl.ds(s * RPT, RPT)],
                    out_hbm.at[c, pl.ds(s * RPT, RPT)])


_agg = functools.partial(
    pl.kernel,
    out_type=jax.ShapeDtypeStruct((NC, NPAD, H1), jnp.float32),
    mesh=plsc.VectorSubcoreMesh(core_axis_name="c", subcore_axis_name="s"),
    compiler_params=pltpu.CompilerParams(use_tc_tiling_on_sc=False),
    scratch_types=[
        pltpu.VMEM((NCHUNK, CH), jnp.int32),
        pltpu.VMEM((NCHUNK, CH), jnp.int32),
        pltpu.VMEM((EPWP,), jnp.float32),
        pltpu.VMEM((NBUF, CH, H1), jnp.float32),
        pltpu.VMEM((NBUF, CH, H1), jnp.float32),
        pltpu.VMEM((RPT, H1), jnp.float32),
        pltpu.VMEM_SHARED((NPAD, H1), jnp.float32),
        pltpu.SemaphoreType.DMA((NBUF,)),
        pltpu.SemaphoreType.DMA((NBUF,)),
    ],
)(_agg_body)


def _proj_body(x_ref, wt_ref, o_ref):
    o_ref[...] = jnp.dot(x_ref[...], wt_ref[...],
                         preferred_element_type=jnp.float32,
                         precision=jax.lax.Precision.HIGHEST)


def _combine_relu_body(p_ref, parts_ref, sw_ref, b_ref, o_ref):
    acc = parts_ref[0] + parts_ref[1]
    o_ref[...] = jnp.maximum(
        p_ref[...] * sw_ref[...] + acc + b_ref[...], 0.0)


def _combine_mm_body(x_ref, parts_ref, sw_ref, wt_ref, b_ref, o_ref):
    h2 = x_ref[...] * sw_ref[...] + parts_ref[0] + parts_ref[1]
    o_ref[...] = jnp.dot(h2, wt_ref[...],
                         preferred_element_type=jnp.float32,
                         precision=jax.lax.Precision.HIGHEST) + b_ref[...]


def _prep_edges(edge_index, edge_weight):
    npad_e = EPWP - EPW
    src = jnp.concatenate(
        [edge_index[0].reshape(NW, EPW),
         jnp.zeros((NW, npad_e), jnp.int32)], axis=1).reshape(NW, NCHUNK, CH)
    dst = jnp.concatenate(
        [edge_index[1].reshape(NW, EPW),
         jnp.full((NW, npad_e), PAD_DST, jnp.int32)],
        axis=1).reshape(NW, NCHUNK, CH)
    w = jnp.concatenate(
        [edge_weight.reshape(NW, EPW),
         jnp.full((NW, npad_e), -1.0, jnp.float32)], axis=1)
    return src, dst, w


def kernel(features, edge_index, edge_weight, self_weight, W1, b1, W2, b2):
    src, dst, w = _prep_edges(edge_index, edge_weight)
    swp1 = self_weight + 1.0

    p1 = pl.pallas_call(
        _proj_body,
        out_shape=jax.ShapeDtypeStruct((N, H1), jnp.float32),
    )(features, W1.T)

    parts1 = _agg(p1, src, dst, w)[:, :N, :]

    x = pl.pallas_call(
        _combine_relu_body,
        out_shape=jax.ShapeDtypeStruct((N, H1), jnp.float32),
    )(p1, parts1, swp1, b1.reshape(1, H1))

    parts2 = _agg(x, src, dst, w)[:, :N, :]

    out = pl.pallas_call(
        _combine_mm_body,
        out_shape=jax.ShapeDtypeStruct((N, W2.shape[0]), jnp.float32),
    )(x, parts2, swp1, W2.T, b2.reshape(1, W2.shape[0]))

    return out


# speedup vs baseline: 1.9398x; 1.5265x over previous
import functools

import jax
import jax.numpy as jnp
from jax import lax
from jax.experimental import pallas as pl
from jax.experimental.pallas import tpu as pltpu, tpu_sc as plsc

N = 10000
E = 320000
D = 128
H1 = 16

NC = 2
NS = 16
NW = NC * NS
EPW = E // NW
CH = 256
EPWP = 10240
NCHUNK = EPWP // CH
NBUF = 4
NSTEP = NCHUNK // NBUF
NPAD = 10240
RPT = NPAD // NS
PAD_DST = NPAD - 8


def _edge_loop(src_v, dst_v, w_v, gbuf, sbuf, tab_sh, acc_sh, dummy_hbm,
               gsem, ssem):
    def _gwait(b):
        pltpu.make_async_copy(dummy_hbm.at[0, pl.ds(0, CH)], gbuf.at[b],
                              gsem.at[b]).wait()

    def _swait(b):
        pltpu.make_async_copy(dummy_hbm.at[0, pl.ds(0, CH)], sbuf.at[b],
                              ssem.at[b]).wait()

    for b in range(NBUF):
        pltpu.async_copy(tab_sh.at[src_v.at[b]], gbuf.at[b], gsem.at[b])

    def _step(i, carry):
        for b in range(NBUF):
            j = i * NBUF + b
            _gwait(b)

            @pl.when(i > 0)
            def _():
                _swait(b)

            def _grp(g, carry2):
                wv16 = w_v[pl.ds(pl.multiple_of(j * CH + g * 16, 16), 16)] + 1.0
                base = g * 16
                for r in range(16):
                    sbuf[b, base + r] = gbuf[b, base + r] * wv16[r]
                return carry2
            lax.fori_loop(0, CH // 16, _grp, 0)

            pltpu.async_copy(sbuf.at[b], acc_sh.at[dst_v.at[j]],
                             ssem.at[b], add=True)

            @pl.when(i < NSTEP - 1)
            def _():
                pltpu.async_copy(tab_sh.at[src_v.at[j + NBUF]], gbuf.at[b],
                                 gsem.at[b])
        return carry
    lax.fori_loop(0, NSTEP, _step, 0)

    for b in range(NBUF):
        _swait(b)


def _self_term_init(c, s, stage_v, acc_v, sw_v, acc_sh):
    @pl.when(c == 0)
    def _():
        def _prod(g, carry):
            swv = sw_v[pl.ds(pl.multiple_of(g * 16, 16), 16)]
            base = g * 16
            for r in range(16):
                acc_v[base + r] = stage_v[base + r] * swv[r]
            return carry
        lax.fori_loop(0, RPT // 16, _prod, 0)

    @pl.when(c != 0)
    def _():
        def _zero(i, carry):
            acc_v[i] = jnp.zeros((16,), jnp.float32)
            return carry
        lax.fori_loop(0, RPT, _zero, 0, unroll=8)

    pltpu.sync_copy(acc_v, acc_sh.at[pl.ds(s * RPT, RPT)])


def _agg1_body(p_hbm, src_hbm, dst_hbm, w_hbm, sw_hbm, out_hbm,
               src_v, dst_v, w_v, gbuf, sbuf, stage_v, acc_v, sw_v,
               tab_sh, acc_sh, gsem, ssem):
    c = lax.axis_index("c")
    s = lax.axis_index("s")
    wid = c * NS + s
    sl = pl.ds(s * RPT, RPT)

    pltpu.sync_copy(src_hbm.at[wid], src_v)
    pltpu.sync_copy(dst_hbm.at[wid], dst_v)
    pltpu.sync_copy(w_hbm.at[wid], w_v)
    pltpu.sync_copy(sw_hbm.at[sl], sw_v)

    pltpu.sync_copy(p_hbm.at[sl], stage_v)
    pltpu.sync_copy(stage_v, tab_sh.at[sl])
    _self_term_init(c, s, stage_v, acc_v, sw_v, acc_sh)
    plsc.subcore_barrier()

    _edge_loop(src_v, dst_v, w_v, gbuf, sbuf, tab_sh, acc_sh, out_hbm,
               gsem, ssem)
    plsc.subcore_barrier()

    pltpu.sync_copy(acc_sh.at[sl], out_hbm.at[c, sl])


def _agg2_body(parts_hbm, src_hbm, dst_hbm, w_hbm, sw_hbm, b1_hbm, out_hbm,
               src_v, dst_v, w_v, gbuf, sbuf, stage_v, acc_v, aux_v, sw_v,
               b1_v, tab_sh, acc_sh, gsem, ssem):
    c = lax.axis_index("c")
    s = lax.axis_index("s")
    wid = c * NS + s
    sl = pl.ds(s * RPT, RPT)

    pltpu.sync_copy(src_hbm.at[wid], src_v)
    pltpu.sync_copy(dst_hbm.at[wid], dst_v)
    pltpu.sync_copy(w_hbm.at[wid], w_v)
    pltpu.sync_copy(sw_hbm.at[sl], sw_v)
    pltpu.sync_copy(b1_hbm, b1_v)

    pltpu.sync_copy(parts_hbm.at[0, sl], stage_v)
    pltpu.sync_copy(parts_hbm.at[1, sl], aux_v)
    b1v = b1_v[...]

    def _xrow(i, carry):
        stage_v[i] = jnp.maximum(stage_v[i] + aux_v[i] + b1v, 0.0)
        return carry
    lax.fori_loop(0, RPT, _xrow, 0, unroll=8)

    pltpu.sync_copy(stage_v, tab_sh.at[sl])
    _self_term_init(c, s, stage_v, acc_v, sw_v, acc_sh)
    plsc.subcore_barrier()

    _edge_loop(src_v, dst_v, w_v, gbuf, sbuf, tab_sh, acc_sh, out_hbm,
               gsem, ssem)
    plsc.subcore_barrier()

    pltpu.sync_copy(acc_sh.at[sl], out_hbm.at[c, sl])


_COMMON_SCRATCH = [
    pltpu.VMEM((NCHUNK, CH), jnp.int32),
    pltpu.VMEM((NCHUNK, CH), jnp.int32),
    pltpu.VMEM((EPWP,), jnp.float32),
    pltpu.VMEM((NBUF, CH, H1), jnp.float32),
    pltpu.VMEM((NBUF, CH, H1), jnp.float32),
    pltpu.VMEM((RPT, H1), jnp.float32),
    pltpu.VMEM((RPT, H1), jnp.float32),
]
_TAIL_SCRATCH = [
    pltpu.VMEM((RPT,), jnp.float32),
    pltpu.VMEM_SHARED((NPAD, H1), jnp.float32),
    pltpu.VMEM_SHARED((NPAD, H1), jnp.float32),
    pltpu.SemaphoreType.DMA((NBUF,)),
    pltpu.SemaphoreType.DMA((NBUF,)),
]

_MESH = plsc.VectorSubcoreMesh(core_axis_name="c", subcore_axis_name="s")

_agg1 = functools.partial(
    pl.kernel,
    out_type=jax.ShapeDtypeStruct((NC, NPAD, H1), jnp.float32),
    mesh=_MESH,
    compiler_params=pltpu.CompilerParams(use_tc_tiling_on_sc=False),
    scratch_types=_COMMON_SCRATCH + _TAIL_SCRATCH,
)(_agg1_body)

_agg2 = functools.partial(
    pl.kernel,
    out_type=jax.ShapeDtypeStruct((NC, NPAD, H1), jnp.float32),
    mesh=_MESH,
    compiler_params=pltpu.CompilerParams(use_tc_tiling_on_sc=False),
    scratch_types=(_COMMON_SCRATCH
                   + [pltpu.VMEM((RPT, H1), jnp.float32),
                      pltpu.VMEM((RPT,), jnp.float32),
                      pltpu.VMEM((H1,), jnp.float32)]
                   + _TAIL_SCRATCH[1:]),
)(_agg2_body)


def _proj_body(x_ref, wt_ref, o_ref):
    o_ref[...] = jnp.dot(x_ref[...], wt_ref[...],
                         preferred_element_type=jnp.float32,
                         precision=jax.lax.Precision.HIGHEST)


def _final_body(parts_ref, wt_ref, b_ref, o_ref):
    h2 = parts_ref[0] + parts_ref[1]
    o_ref[...] = jnp.dot(h2, wt_ref[...],
                         preferred_element_type=jnp.float32,
                         precision=jax.lax.Precision.HIGHEST) + b_ref[...]


def _prep_edges(edge_index, edge_weight):
    npad_e = EPWP - EPW
    src = jnp.concatenate(
        [edge_index[0].reshape(NW, EPW),
         jnp.zeros((NW, npad_e), jnp.int32)], axis=1).reshape(NW, NCHUNK, CH)
    dst = jnp.concatenate(
        [edge_index[1].reshape(NW, EPW),
         jnp.full((NW, npad_e), PAD_DST, jnp.int32)],
        axis=1).reshape(NW, NCHUNK, CH)
    w = jnp.concatenate(
        [edge_weight.reshape(NW, EPW),
         jnp.full((NW, npad_e), -1.0, jnp.float32)], axis=1)
    return src, dst, w


def kernel(features, edge_index, edge_weight, self_weight, W1, b1, W2, b2):
    src, dst, w = _prep_edges(edge_index, edge_weight)
    swpad = jnp.pad(self_weight.reshape(N) + 1.0, (0, NPAD - N))
    fpad = jnp.pad(features, ((0, NPAD - N), (0, 0)))

    p1 = pl.pallas_call(
        _proj_body,
        out_shape=jax.ShapeDtypeStruct((NPAD, H1), jnp.float32),
    )(fpad, W1.T)

    parts1 = _agg1(p1, src, dst, w, swpad)
    parts2 = _agg2(parts1, src, dst, w, swpad, b1)

    out = pl.pallas_call(
        _final_body,
        out_shape=jax.ShapeDtypeStruct((NPAD, W2.shape[0]), jnp.float32),
    )(parts2, W2.T, b2.reshape(1, W2.shape[0]))

    return out[:N]


# speedup vs baseline: 2.3301x; 1.2012x over previous
import functools

import jax
import jax.numpy as jnp
from jax import lax
from jax.experimental import pallas as pl
from jax.experimental.pallas import tpu as pltpu, tpu_sc as plsc

N = 10000
E = 320000
D = 128
H1 = 16

NC = 2
NS = 16
NW = NC * NS
EPW = E // NW
CH = 200
NCHUNK = EPW // CH
NBUF = 5
NSTEP = NCHUNK // NBUF
NPAD = 10240
RPT = NPAD // NS


def _edge_loop(src_v, dst_v, w_v, gbuf, sbuf, tab_sh, acc_sh, dummy_hbm,
               gsem, ssem):
    def _gwait(b):
        pltpu.make_async_copy(dummy_hbm.at[0, pl.ds(0, CH)], gbuf.at[b],
                              gsem.at[b]).wait()

    def _swait(b):
        pltpu.make_async_copy(dummy_hbm.at[0, pl.ds(0, CH)], sbuf.at[b],
                              ssem.at[b]).wait()

    def _idx(j):
        return pl.ds(pl.multiple_of(j * CH, 8), CH)

    for b in range(NBUF):
        pltpu.async_copy(tab_sh.at[src_v.at[_idx(b)]], gbuf.at[b], gsem.at[b])

    def _step(i, carry):
        for b in range(NBUF):
            j = i * NBUF + b
            _gwait(b)

            @pl.when(i > 0)
            def _():
                _swait(b)

            def _grp(g, carry2):
                wv16 = w_v[pl.ds(pl.multiple_of(j * CH + g * 16, 8), 16)] + 1.0
                base = g * 16
                for r in range(16):
                    sbuf[b, base + r] = gbuf[b, base + r] * wv16[r]
                return carry2
            lax.fori_loop(0, CH // 16, _grp, 0)

            pltpu.async_copy(sbuf.at[b], acc_sh.at[dst_v.at[_idx(j)]],
                             ssem.at[b], add=True)

            @pl.when(i < NSTEP - 1)
            def _():
                pltpu.async_copy(tab_sh.at[src_v.at[_idx(j + NBUF)]],
                                 gbuf.at[b], gsem.at[b])
        return carry
    lax.fori_loop(0, NSTEP, _step, 0)

    for b in range(NBUF):
        _swait(b)


def _self_term_init(c, s, stage_v, acc_v, sw_v, acc_sh):
    @pl.when(c == 0)
    def _():
        def _prod(g, carry):
            swv = sw_v[pl.ds(pl.multiple_of(g * 16, 16), 16)]
            base = g * 16
            for r in range(16):
                acc_v[base + r] = stage_v[base + r] * swv[r]
            return carry
        lax.fori_loop(0, RPT // 16, _prod, 0)

    @pl.when(c != 0)
    def _():
        def _zero(i, carry):
            acc_v[i] = jnp.zeros((16,), jnp.float32)
            return carry
        lax.fori_loop(0, RPT, _zero, 0, unroll=8)

    pltpu.sync_copy(acc_v, acc_sh.at[pl.ds(s * RPT, RPT)])


def _agg1_body(p_hbm, eidx_hbm, w_hbm, sw_hbm, out_hbm,
               src_v, dst_v, w_v, gbuf, sbuf, stage_v, acc_v, sw_v,
               tab_sh, acc_sh, gsem, ssem):
    c = lax.axis_index("c")
    s = lax.axis_index("s")
    wid = c * NS + s
    sl = pl.ds(s * RPT, RPT)
    esl = pl.ds(pl.multiple_of(wid * EPW, 16), EPW)
    dsl = pl.ds(pl.multiple_of(E + wid * EPW, 16), EPW)

    pltpu.sync_copy(eidx_hbm.at[esl], src_v)
    pltpu.sync_copy(eidx_hbm.at[dsl], dst_v)
    pltpu.sync_copy(w_hbm.at[esl], w_v)
    pltpu.sync_copy(sw_hbm.at[sl], sw_v)

    pltpu.sync_copy(p_hbm.at[sl], stage_v)
    pltpu.sync_copy(stage_v, tab_sh.at[sl])
    _self_term_init(c, s, stage_v, acc_v, sw_v, acc_sh)
    plsc.subcore_barrier()

    _edge_loop(src_v, dst_v, w_v, gbuf, sbuf, tab_sh, acc_sh, out_hbm,
               gsem, ssem)
    plsc.subcore_barrier()

    pltpu.sync_copy(acc_sh.at[sl], out_hbm.at[c, sl])


def _agg2_body(parts_hbm, eidx_hbm, w_hbm, sw_hbm, b1_hbm, out_hbm,
               src_v, dst_v, w_v, gbuf, sbuf, stage_v, acc_v, aux_v, sw_v,
               b1_v, tab_sh, acc_sh, gsem, ssem):
    c = lax.axis_index("c")
    s = lax.axis_index("s")
    wid = c * NS + s
    sl = pl.ds(s * RPT, RPT)
    esl = pl.ds(pl.multiple_of(wid * EPW, 16), EPW)
    dsl = pl.ds(pl.multiple_of(E + wid * EPW, 16), EPW)

    pltpu.sync_copy(eidx_hbm.at[esl], src_v)
    pltpu.sync_copy(eidx_hbm.at[dsl], dst_v)
    pltpu.sync_copy(w_hbm.at[esl], w_v)
    pltpu.sync_copy(sw_hbm.at[sl], sw_v)
    pltpu.sync_copy(b1_hbm, b1_v)

    pltpu.sync_copy(parts_hbm.at[0, sl], stage_v)
    pltpu.sync_copy(parts_hbm.at[1, sl], aux_v)
    b1v = b1_v[...]

    def _xrow(i, carry):
        stage_v[i] = jnp.maximum(stage_v[i] + aux_v[i] + b1v, 0.0)
        return carry
    lax.fori_loop(0, RPT, _xrow, 0, unroll=8)

    pltpu.sync_copy(stage_v, tab_sh.at[sl])
    _self_term_init(c, s, stage_v, acc_v, sw_v, acc_sh)
    plsc.subcore_barrier()

    _edge_loop(src_v, dst_v, w_v, gbuf, sbuf, tab_sh, acc_sh, out_hbm,
               gsem, ssem)
    plsc.subcore_barrier()

    pltpu.sync_copy(acc_sh.at[sl], out_hbm.at[c, sl])


_COMMON_SCRATCH = [
    pltpu.VMEM((EPW,), jnp.int32),
    pltpu.VMEM((EPW,), jnp.int32),
    pltpu.VMEM((EPW,), jnp.float32),
    pltpu.VMEM((NBUF, CH, H1), jnp.float32),
    pltpu.VMEM((NBUF, CH, H1), jnp.float32),
    pltpu.VMEM((RPT, H1), jnp.float32),
    pltpu.VMEM((RPT, H1), jnp.float32),
]
_TAIL_SCRATCH = [
    pltpu.VMEM((RPT,), jnp.float32),
    pltpu.VMEM_SHARED((NPAD, H1), jnp.float32),
    pltpu.VMEM_SHARED((NPAD, H1), jnp.float32),
    pltpu.SemaphoreType.DMA((NBUF,)),
    pltpu.SemaphoreType.DMA((NBUF,)),
]

_MESH = plsc.VectorSubcoreMesh(core_axis_name="c", subcore_axis_name="s",
                               num_cores=NC, num_subcores=NS)

_agg1 = functools.partial(
    pl.kernel,
    out_type=jax.ShapeDtypeStruct((NC, NPAD, H1), jnp.float32),
    mesh=_MESH,
    compiler_params=pltpu.CompilerParams(use_tc_tiling_on_sc=False),
    scratch_types=_COMMON_SCRATCH + _TAIL_SCRATCH,
)(_agg1_body)

_agg2 = functools.partial(
    pl.kernel,
    out_type=jax.ShapeDtypeStruct((NC, NPAD, H1), jnp.float32),
    mesh=_MESH,
    compiler_params=pltpu.CompilerParams(use_tc_tiling_on_sc=False),
    scratch_types=(_COMMON_SCRATCH
                   + [pltpu.VMEM((RPT, H1), jnp.float32),
                      pltpu.VMEM((RPT,), jnp.float32),
                      pltpu.VMEM((H1,), jnp.float32)]
                   + _TAIL_SCRATCH[1:]),
)(_agg2_body)


def _proj_body(x_ref, wt_ref, o_ref):
    o_ref[...] = jnp.dot(x_ref[...], wt_ref[...],
                         preferred_element_type=jnp.float32,
                         precision=jax.lax.Precision.HIGHEST)


def _final_body(parts_ref, wt_ref, b_ref, o_ref):
    h2 = parts_ref[0] + parts_ref[1]
    o_ref[...] = jnp.dot(h2, wt_ref[...],
                         preferred_element_type=jnp.float32,
                         precision=jax.lax.Precision.HIGHEST) + b_ref[...]


def kernel(features, edge_index, edge_weight, self_weight, W1, b1, W2, b2):
    eidx = edge_index.reshape(2 * E)
    w = edge_weight.reshape(E)
    swpad = jnp.pad(self_weight.reshape(N) + 1.0, (0, NPAD - N))
    fpad = jnp.pad(features, ((0, NPAD - N), (0, 0)))

    p1 = pl.pallas_call(
        _proj_body,
        out_shape=jax.ShapeDtypeStruct((NPAD, H1), jnp.float32),
    )(fpad, W1.T)

    parts1 = _agg1(p1, eidx, w, swpad)
    parts2 = _agg2(parts1, eidx, w, swpad, b1)

    out = pl.pallas_call(
        _final_body,
        out_shape=jax.ShapeDtypeStruct((NPAD, W2.shape[0]), jnp.float32),
    )(parts2, W2.T, b2.reshape(1, W2.shape[0]))

    return out[:N]
